# Initial kernel scaffold; baseline (speedup 1.0000x reference)
import jax
import jax.numpy as jnp
from jax.experimental import pallas as pl


def kernel(X, edge_index, edge_weight, emb_table, gcn1_W, gcn1_b, bn1_gamma, bn1_beta, gcn2_W, gcn2_b, bn2_gamma, bn2_beta, Wq, Wk, Wv, Wagg, alpha):
    raise NotImplementedError("write your pallas kernel here")


# speedup vs baseline: 23.8994x; 23.8994x over previous
import functools

import jax
import jax.numpy as jnp
from jax import lax
from jax.experimental import pallas as pl
from jax.experimental.pallas import tpu as pltpu
from jax.experimental.pallas import tpu_sc as plsc

TT = 4
N = 2048
DD = 128
NH = 4
EE = 131072


QROWS = 512
QWORDS = QROWS * N
EPT = EE // 16
ZWORDS = 16384
TILE_QW = QWORDS // 16


def _make_adj():
    mesh = plsc.VectorSubcoreMesh(core_axis_name="c", subcore_axis_name="s")

    @functools.partial(
        pl.kernel,
        mesh=mesh,
        out_type=jax.ShapeDtypeStruct((N * N,), jnp.float32),
        scratch_types=[
            pltpu.VMEM((EPT,), jnp.int32),
            pltpu.VMEM((EPT,), jnp.int32),
            pltpu.VMEM((EPT,), jnp.float32),
            pltpu.VMEM((EPT,), jnp.int32),
            pltpu.VMEM((EPT,), jnp.float32),
            pltpu.VMEM((ZWORDS,), jnp.float32),
            pltpu.VMEM_SHARED((QWORDS,), jnp.float32),
        ],
    )
    def adj(edge_hbm, ew_hbm, out_hbm, r_v, c_v, ew_v, idx_v, val_v, zero_v, qmem):
        cid = lax.axis_index("c")
        sid = lax.axis_index("s")
        ebase = sid * EPT
        pltpu.sync_copy(edge_hbm.at[0, pl.ds(ebase, EPT)], r_v)
        pltpu.sync_copy(edge_hbm.at[1, pl.ds(ebase, EPT)], c_v)
        pltpu.sync_copy(ew_hbm.at[pl.ds(ebase, EPT)], ew_v)

        def zbody(i, _):
            zero_v[pl.ds(i * 16, 16)] = jnp.zeros((16,), jnp.float32)
            return 0
        lax.fori_loop(0, ZWORDS // 16, zbody, 0)

        for q in range(2):
            q0 = (cid * 2 + q) * QROWS
            for z in range(TILE_QW // ZWORDS):
                pltpu.sync_copy(
                    zero_v, qmem.at[pl.ds(sid * TILE_QW + z * ZWORDS, ZWORDS)])
            plsc.subcore_barrier()

            def ebody(j, _):
                for kk in range(8):
                    sl = pl.ds(j * 128 + kk * 16, 16)
                    rel = c_v[sl] - q0
                    ok = (rel >= 0) & (rel < QROWS)
                    idx_v[sl] = jnp.where(ok, rel * N + r_v[sl], 0)
                    val_v[sl] = jnp.where(ok, ew_v[sl], 0.0)
                return 0
            lax.fori_loop(0, EPT // 128, ebody, 0)
            pltpu.sync_copy(val_v, qmem.at[idx_v], add=True)
            plsc.subcore_barrier()
            pltpu.sync_copy(
                qmem.at[pl.ds(sid * TILE_QW, TILE_QW)],
                out_hbm.at[pl.ds(q0 * N + sid * TILE_QW, TILE_QW)])
            plsc.subcore_barrier()

    return adj


_adj = _make_adj()


_HI = lax.Precision.HIGHEST


def _dinv_body(a_ref, o_ref):
    s = jnp.sum(a_ref[...], axis=1, keepdims=True) + 1.0
    o_ref[...] = jnp.broadcast_to(lax.rsqrt(s), (N // 8, DD))


_dinv_call = pl.pallas_call(
    _dinv_body,
    grid=(8,),
    in_specs=[pl.BlockSpec((N // 8, N), lambda i: (i, 0))],
    out_specs=pl.BlockSpec((N // 8, DD), lambda i: (i, 0)),
    out_shape=jax.ShapeDtypeStruct((N, DD), jnp.float32),
)


def _gcn_body(x_ref, w_ref, b_ref, a_ref, dv_ref, o_ref):
    dv = dv_ref[...]
    h = jnp.dot(x_ref[0], w_ref[...], preferred_element_type=jnp.float32,
                precision=_HI)
    hs = h * dv
    m = jnp.dot(a_ref[...], hs, preferred_element_type=jnp.float32,
                precision=_HI) + hs
    o_ref[0] = m * dv + b_ref[0:1]


_gcn_call = pl.pallas_call(
    _gcn_body,
    grid=(TT,),
    in_specs=[
        pl.BlockSpec((1, N, DD), lambda t: (t, 0, 0)),
        pl.BlockSpec((DD, DD), lambda t: (0, 0)),
        pl.BlockSpec((8, DD), lambda t: (0, 0)),
        pl.BlockSpec((N, N), lambda t: (0, 0)),
        pl.BlockSpec((N, DD), lambda t: (0, 0)),
    ],
    out_specs=pl.BlockSpec((1, N, DD), lambda t: (t, 0, 0)),
    out_shape=jax.ShapeDtypeStruct((TT, N, DD), jnp.float32),
)


def _bnrelu_body(g_ref, gm_ref, bt_ref, o_ref):
    g = g_ref[...]
    g2 = g.reshape(TT * N, DD)
    mean = jnp.mean(g2, axis=0, keepdims=True)
    var = jnp.mean((g2 - mean) ** 2, axis=0, keepdims=True)
    scale = gm_ref[0:1] * lax.rsqrt(var + 1e-5)
    shift = bt_ref[0:1] - mean * scale
    o_ref[...] = jnp.maximum(g * scale.reshape(1, 1, DD)
                             + shift.reshape(1, 1, DD), 0.0)


_bnrelu_call = pl.pallas_call(
    _bnrelu_body,
    out_shape=jax.ShapeDtypeStruct((TT, N, DD), jnp.float32),
)


RB = 256
NRB = N // RB
_SCALE = 1.0 / float(DD) ** 0.5


def _attn_body(h_ref, wq_ref, wk_ref, wv_ref, wagg_ref, al_ref, emb_ref,
               o_ref, xq, acc):
    hh = pl.program_id(1)
    x = h_ref[0]
    xq[...] = jnp.dot(x, wq_ref[...], preferred_element_type=jnp.float32,
                      precision=_HI)
    xk = jnp.dot(x, wk_ref[...], preferred_element_type=jnp.float32,
                 precision=_HI)
    xv = jnp.dot(x, wv_ref[...], preferred_element_type=jnp.float32,
                 precision=_HI)

    @pl.when(hh == 0)
    def _():
        acc[...] = jnp.zeros_like(acc)

    def rb_body(i, _):
        qb = xq[pl.ds(i * RB, RB), :]
        s = lax.dot_general(qb, xk, (((1,), (1,)), ((), ())),
                            preferred_element_type=jnp.float32,
                            precision=_HI) * _SCALE
        rows = i * RB + lax.broadcasted_iota(jnp.int32, (RB, N), 0)
        cols = lax.broadcasted_iota(jnp.int32, (RB, N), 1)
        s = jnp.where(cols <= rows, s, -jnp.inf)
        m = jnp.max(s, axis=1, keepdims=True)
        p = jnp.exp(s - m)
        p = p / jnp.sum(p, axis=1, keepdims=True)
        ob = jnp.dot(p, xv, preferred_element_type=jnp.float32, precision=_HI)
        acc[pl.ds(i * RB, RB), :] += ob
        return 0

    lax.fori_loop(0, NRB, rb_body, 0)

    @pl.when(hh == NH - 1)
    def _():
        mh = acc[...] * (1.0 / NH)
        agg = jnp.dot(mh, wagg_ref[...], preferred_element_type=jnp.float32,
                      precision=_HI)
        a = al_ref[...]
        o_ref[0] = (1.0 - a) * emb_ref[...] + a * agg


_attn_call = pl.pallas_call(
    _attn_body,
    grid=(TT, NH),
    in_specs=[
        pl.BlockSpec((1, N, DD), lambda t, h: (t, 0, 0)),
        pl.BlockSpec((DD, DD), lambda t, h: (0, h)),
        pl.BlockSpec((DD, DD), lambda t, h: (0, h)),
        pl.BlockSpec((DD, DD), lambda t, h: (0, h)),
        pl.BlockSpec((DD, DD), lambda t, h: (0, 0)),
        pl.BlockSpec((N, DD), lambda t, h: (0, 0)),
        pl.BlockSpec((N, DD), lambda t, h: (0, 0)),
    ],
    out_specs=pl.BlockSpec((1, N, DD), lambda t, h: (t, 0, 0)),
    out_shape=jax.ShapeDtypeStruct((TT, N, DD), jnp.float32),
    scratch_shapes=[
        pltpu.VMEM((N, DD), jnp.float32),
        pltpu.VMEM((N, DD), jnp.float32),
    ],
)


def kernel(X, edge_index, edge_weight, emb_table, gcn1_W, gcn1_b, bn1_gamma,
           bn1_beta, gcn2_W, gcn2_b, bn2_gamma, bn2_beta, Wq, Wk, Wv, Wagg,
           alpha):
    A = _adj(edge_index.astype(jnp.int32), edge_weight).reshape(N, N)
    dinv = _dinv_call(A)

    def row8(v):
        return jnp.broadcast_to(v.reshape(1, DD), (8, DD))

    g1 = _gcn_call(X, gcn1_W, row8(gcn1_b), A, dinv)
    h1 = _bnrelu_call(g1, row8(bn1_gamma), row8(bn1_beta))
    g2 = _gcn_call(h1, gcn2_W, row8(gcn2_b), A, dinv)
    h2 = _bnrelu_call(g2, row8(bn2_gamma), row8(bn2_beta))

    alpha2 = jnp.broadcast_to(alpha, (N, DD))
    out = _attn_call(h2, Wq, Wk, Wv, Wagg, alpha2, emb_table)
    return out


# speedup vs baseline: 30.4094x; 1.2724x over previous
import functools

import jax
import jax.numpy as jnp
from jax import lax
from jax.experimental import pallas as pl
from jax.experimental.pallas import tpu as pltpu
from jax.experimental.pallas import tpu_sc as plsc

TT = 4
N = 2048
DD = 128
NH = 4
EE = 131072


QROWS = 512
QWORDS = QROWS * N
EPT = EE // 16
ZWORDS = 16384
TILE_QW = QWORDS // 16


def _make_adj():
    mesh = plsc.VectorSubcoreMesh(core_axis_name="c", subcore_axis_name="s")

    @functools.partial(
        pl.kernel,
        mesh=mesh,
        out_type=jax.ShapeDtypeStruct((N * N,), jnp.float32),
        scratch_types=[
            pltpu.VMEM((EPT,), jnp.int32),
            pltpu.VMEM((EPT,), jnp.int32),
            pltpu.VMEM((EPT,), jnp.float32),
            pltpu.VMEM((EPT,), jnp.int32),
            pltpu.VMEM((EPT,), jnp.float32),
            pltpu.VMEM((ZWORDS,), jnp.float32),
            pltpu.VMEM_SHARED((QWORDS,), jnp.float32),
        ],
    )
    def adj(edge_hbm, ew_hbm, out_hbm, r_v, c_v, ew_v, idx_v, val_v, zero_v, qmem):
        cid = lax.axis_index("c")
        sid = lax.axis_index("s")
        ebase = sid * EPT
        pltpu.sync_copy(edge_hbm.at[0, pl.ds(ebase, EPT)], r_v)
        pltpu.sync_copy(edge_hbm.at[1, pl.ds(ebase, EPT)], c_v)
        pltpu.sync_copy(ew_hbm.at[pl.ds(ebase, EPT)], ew_v)

        def zbody(i, _):
            zero_v[pl.ds(i * 16, 16)] = jnp.zeros((16,), jnp.float32)
            return 0
        lax.fori_loop(0, ZWORDS // 16, zbody, 0)

        for q in range(2):
            q0 = (cid * 2 + q) * QROWS
            for z in range(TILE_QW // ZWORDS):
                pltpu.sync_copy(
                    zero_v, qmem.at[pl.ds(sid * TILE_QW + z * ZWORDS, ZWORDS)])
            plsc.subcore_barrier()

            def ebody(j, _):
                for kk in range(8):
                    sl = pl.ds(j * 128 + kk * 16, 16)
                    flat = c_v[sl] * N + r_v[sl]
                    rel = c_v[sl] - q0
                    ok = (rel >= 0) & (rel < QROWS)
                    idx_v[sl] = jnp.where(ok, rel * N + r_v[sl],
                                          flat & (QWORDS - 1))
                    val_v[sl] = jnp.where(ok, ew_v[sl], 0.0)
                return 0
            lax.fori_loop(0, EPT // 128, ebody, 0)
            pltpu.sync_copy(val_v, qmem.at[idx_v], add=True)
            plsc.subcore_barrier()
            pltpu.sync_copy(
                qmem.at[pl.ds(sid * TILE_QW, TILE_QW)],
                out_hbm.at[pl.ds(q0 * N + sid * TILE_QW, TILE_QW)])
            plsc.subcore_barrier()

    return adj


_adj = _make_adj()


_HI = lax.Precision.HIGHEST


def _dinv_body(a_ref, o_ref):
    s = jnp.sum(a_ref[...], axis=1, keepdims=True) + 1.0
    o_ref[...] = jnp.broadcast_to(lax.rsqrt(s), (N // 8, DD))


_dinv_call = pl.pallas_call(
    _dinv_body,
    grid=(8,),
    in_specs=[pl.BlockSpec((N // 8, N), lambda i: (i, 0))],
    out_specs=pl.BlockSpec((N // 8, DD), lambda i: (i, 0)),
    out_shape=jax.ShapeDtypeStruct((N, DD), jnp.float32),
)


def _gcn_body(x_ref, w_ref, b_ref, a_ref, dv_ref, o_ref):
    dv = dv_ref[...]
    h = jnp.dot(x_ref[0], w_ref[...], preferred_element_type=jnp.float32,
                precision=_HI)
    hs = h * dv
    m = jnp.dot(a_ref[...], hs, preferred_element_type=jnp.float32,
                precision=_HI) + hs
    o_ref[0] = m * dv + b_ref[0:1]


_gcn_call = pl.pallas_call(
    _gcn_body,
    grid=(TT,),
    in_specs=[
        pl.BlockSpec((1, N, DD), lambda t: (t, 0, 0)),
        pl.BlockSpec((DD, DD), lambda t: (0, 0)),
        pl.BlockSpec((8, DD), lambda t: (0, 0)),
        pl.BlockSpec((N, N), lambda t: (0, 0)),
        pl.BlockSpec((N, DD), lambda t: (0, 0)),
    ],
    out_specs=pl.BlockSpec((1, N, DD), lambda t: (t, 0, 0)),
    out_shape=jax.ShapeDtypeStruct((TT, N, DD), jnp.float32),
)


def _bnrelu_body(g_ref, gm_ref, bt_ref, o_ref):
    g = g_ref[...]
    g2 = g.reshape(TT * N, DD)
    mean = jnp.mean(g2, axis=0, keepdims=True)
    var = jnp.mean((g2 - mean) ** 2, axis=0, keepdims=True)
    scale = gm_ref[0:1] * lax.rsqrt(var + 1e-5)
    shift = bt_ref[0:1] - mean * scale
    o_ref[...] = jnp.maximum(g * scale.reshape(1, 1, DD)
                             + shift.reshape(1, 1, DD), 0.0)


_bnrelu_call = pl.pallas_call(
    _bnrelu_body,
    out_shape=jax.ShapeDtypeStruct((TT, N, DD), jnp.float32),
)


RB = 256
NRB = N // RB
_SCALE = 1.0 / float(DD) ** 0.5


def _attn_body(h_ref, wq_ref, wk_ref, wv_ref, wagg_ref, al_ref, emb_ref,
               o_ref, xq, acc):
    hh = pl.program_id(1)
    x = h_ref[0]
    xq[...] = jnp.dot(x, wq_ref[...], preferred_element_type=jnp.float32,
                      precision=_HI)
    xk = jnp.dot(x, wk_ref[...], preferred_element_type=jnp.float32,
                 precision=_HI)
    xv = jnp.dot(x, wv_ref[...], preferred_element_type=jnp.float32,
                 precision=_HI)

    @pl.when(hh == 0)
    def _():
        acc[...] = jnp.zeros_like(acc)

    def rb_body(i, _):
        qb = xq[pl.ds(i * RB, RB), :]
        s = lax.dot_general(qb, xk, (((1,), (1,)), ((), ())),
                            preferred_element_type=jnp.float32,
                            precision=_HI) * _SCALE
        rows = i * RB + lax.broadcasted_iota(jnp.int32, (RB, N), 0)
        cols = lax.broadcasted_iota(jnp.int32, (RB, N), 1)
        s = jnp.where(cols <= rows, s, -jnp.inf)
        m = jnp.max(s, axis=1, keepdims=True)
        p = jnp.exp(s - m)
        p = p / jnp.sum(p, axis=1, keepdims=True)
        ob = jnp.dot(p, xv, preferred_element_type=jnp.float32, precision=_HI)
        acc[pl.ds(i * RB, RB), :] += ob
        return 0

    lax.fori_loop(0, NRB, rb_body, 0)

    @pl.when(hh == NH - 1)
    def _():
        mh = acc[...] * (1.0 / NH)
        agg = jnp.dot(mh, wagg_ref[...], preferred_element_type=jnp.float32,
                      precision=_HI)
        a = al_ref[...]
        o_ref[0] = (1.0 - a) * emb_ref[...] + a * agg


_attn_call = pl.pallas_call(
    _attn_body,
    grid=(TT, NH),
    in_specs=[
        pl.BlockSpec((1, N, DD), lambda t, h: (t, 0, 0)),
        pl.BlockSpec((DD, DD), lambda t, h: (0, h)),
        pl.BlockSpec((DD, DD), lambda t, h: (0, h)),
        pl.BlockSpec((DD, DD), lambda t, h: (0, h)),
        pl.BlockSpec((DD, DD), lambda t, h: (0, 0)),
        pl.BlockSpec((N, DD), lambda t, h: (0, 0)),
        pl.BlockSpec((N, DD), lambda t, h: (0, 0)),
    ],
    out_specs=pl.BlockSpec((1, N, DD), lambda t, h: (t, 0, 0)),
    out_shape=jax.ShapeDtypeStruct((TT, N, DD), jnp.float32),
    scratch_shapes=[
        pltpu.VMEM((N, DD), jnp.float32),
        pltpu.VMEM((N, DD), jnp.float32),
    ],
)


def kernel(X, edge_index, edge_weight, emb_table, gcn1_W, gcn1_b, bn1_gamma,
           bn1_beta, gcn2_W, gcn2_b, bn2_gamma, bn2_beta, Wq, Wk, Wv, Wagg,
           alpha):
    A = _adj(edge_index.astype(jnp.int32), edge_weight).reshape(N, N)
    dinv = _dinv_call(A)

    def row8(v):
        return jnp.broadcast_to(v.reshape(1, DD), (8, DD))

    g1 = _gcn_call(X, gcn1_W, row8(gcn1_b), A, dinv)
    h1 = _bnrelu_call(g1, row8(bn1_gamma), row8(bn1_beta))
    g2 = _gcn_call(h1, gcn2_W, row8(gcn2_b), A, dinv)
    h2 = _bnrelu_call(g2, row8(bn2_gamma), row8(bn2_beta))

    alpha2 = jnp.broadcast_to(alpha, (N, DD))
    out = _attn_call(h2, Wq, Wk, Wv, Wagg, alpha2, emb_table)
    return out


# speedup vs baseline: 62.6606x; 2.0606x over previous
import functools

import jax
import jax.numpy as jnp
from jax import lax
from jax.experimental import pallas as pl
from jax.experimental.pallas import tpu as pltpu
from jax.experimental.pallas import tpu_sc as plsc

TT = 4
N = 2048
DD = 128
NH = 4
EE = 131072


QROWS = 512
QWORDS = QROWS * N
EPT = EE // 16
ZWORDS = 16384
TILE_QW = QWORDS // 16


def _make_adj():
    mesh = plsc.VectorSubcoreMesh(core_axis_name="c", subcore_axis_name="s")

    @functools.partial(
        pl.kernel,
        mesh=mesh,
        out_type=jax.ShapeDtypeStruct((N * N,), jnp.float32),
        scratch_types=[
            pltpu.VMEM((EPT,), jnp.int32),
            pltpu.VMEM((EPT,), jnp.int32),
            pltpu.VMEM((EPT,), jnp.float32),
            pltpu.VMEM((EPT,), jnp.int32),
            pltpu.VMEM((EPT,), jnp.float32),
            pltpu.VMEM((ZWORDS,), jnp.float32),
            pltpu.VMEM_SHARED((QWORDS,), jnp.float32),
        ],
    )
    def adj(edge_hbm, ew_hbm, out_hbm, r_v, c_v, ew_v, idx_v, val_v, zero_v, qmem):
        cid = lax.axis_index("c")
        sid = lax.axis_index("s")
        ebase = sid * EPT
        pltpu.sync_copy(edge_hbm.at[0, pl.ds(ebase, EPT)], r_v)
        pltpu.sync_copy(edge_hbm.at[1, pl.ds(ebase, EPT)], c_v)
        pltpu.sync_copy(ew_hbm.at[pl.ds(ebase, EPT)], ew_v)

        def zbody(i, _):
            zero_v[pl.ds(i * 16, 16)] = jnp.zeros((16,), jnp.float32)
            return 0
        lax.fori_loop(0, ZWORDS // 16, zbody, 0)

        for q in range(2):
            q0 = (cid * 2 + q) * QROWS
            for z in range(TILE_QW // ZWORDS):
                pltpu.sync_copy(
                    zero_v, qmem.at[pl.ds(sid * TILE_QW + z * ZWORDS, ZWORDS)])
            plsc.subcore_barrier()

            def ebody(j, _):
                for kk in range(8):
                    sl = pl.ds(j * 128 + kk * 16, 16)
                    flat = c_v[sl] * N + r_v[sl]
                    rel = c_v[sl] - q0
                    ok = (rel >= 0) & (rel < QROWS)
                    idx_v[sl] = jnp.where(ok, rel * N + r_v[sl],
                                          flat & (QWORDS - 1))
                    val_v[sl] = jnp.where(ok, ew_v[sl], 0.0)
                return 0
            lax.fori_loop(0, EPT // 128, ebody, 0)
            pltpu.sync_copy(val_v, qmem.at[idx_v], add=True)
            plsc.subcore_barrier()
            pltpu.sync_copy(
                qmem.at[pl.ds(sid * TILE_QW, TILE_QW)],
                out_hbm.at[pl.ds(q0 * N + sid * TILE_QW, TILE_QW)])
            plsc.subcore_barrier()

    return adj


_adj = _make_adj()


_HI = lax.Precision.HIGHEST


def _dinv_body(a_ref, o_ref):
    s = jnp.sum(a_ref[...], axis=1, keepdims=True) + 1.0
    o_ref[...] = jnp.broadcast_to(lax.rsqrt(s), (N // 8, DD))


_dinv_call = pl.pallas_call(
    _dinv_body,
    grid=(8,),
    in_specs=[pl.BlockSpec((N // 8, N), lambda i: (i, 0))],
    out_specs=pl.BlockSpec((N // 8, DD), lambda i: (i, 0)),
    out_shape=jax.ShapeDtypeStruct((N, DD), jnp.float32),
)


def _hs_body(x_ref, w_ref, dv_ref, o_ref):
    dv = dv_ref[...]
    for t in range(TT):
        o_ref[t] = jnp.dot(x_ref[t], w_ref[...],
                           preferred_element_type=jnp.float32,
                           precision=_HI) * dv


_hs_call = pl.pallas_call(
    _hs_body,
    out_shape=jax.ShapeDtypeStruct((TT, N, DD), jnp.float32),
)


def _bnhs_body(g_ref, gm_ref, bt_ref, w_ref, dv_ref, o_ref):
    g2 = g_ref[...].reshape(TT * N, DD)
    mean = jnp.mean(g2, axis=0, keepdims=True)
    var = jnp.mean((g2 - mean) ** 2, axis=0, keepdims=True)
    scale = gm_ref[0:1] * lax.rsqrt(var + 1e-5)
    shift = bt_ref[0:1] - mean * scale
    dv = dv_ref[...]
    for t in range(TT):
        h = jnp.maximum(g_ref[t] * scale + shift, 0.0)
        o_ref[t] = jnp.dot(h, w_ref[...], preferred_element_type=jnp.float32,
                           precision=_HI) * dv


_bnhs_call = pl.pallas_call(
    _bnhs_body,
    out_shape=jax.ShapeDtypeStruct((TT, N, DD), jnp.float32),
)


_RBG = 256


def _gcnmm_body(a_ref, hs_ref, hsb_ref, dvb_ref, b_ref, o_ref):
    dvb = dvb_ref[...]
    for t in range(TT):
        m = jnp.dot(a_ref[...], hs_ref[t], preferred_element_type=jnp.float32,
                    precision=_HI) + hsb_ref[t]
        o_ref[t] = m * dvb + b_ref[0:1]


_gcnmm_call = pl.pallas_call(
    _gcnmm_body,
    grid=(N // _RBG,),
    in_specs=[
        pl.BlockSpec((_RBG, N), lambda i: (i, 0)),
        pl.BlockSpec((TT, N, DD), lambda i: (0, 0, 0)),
        pl.BlockSpec((TT, _RBG, DD), lambda i: (0, i, 0)),
        pl.BlockSpec((_RBG, DD), lambda i: (i, 0)),
        pl.BlockSpec((8, DD), lambda i: (0, 0)),
    ],
    out_specs=pl.BlockSpec((TT, _RBG, DD), lambda i: (0, i, 0)),
    out_shape=jax.ShapeDtypeStruct((TT, N, DD), jnp.float32),
)


def _bnrelu_body(g_ref, gm_ref, bt_ref, o_ref):
    g = g_ref[...]
    g2 = g.reshape(TT * N, DD)
    mean = jnp.mean(g2, axis=0, keepdims=True)
    var = jnp.mean((g2 - mean) ** 2, axis=0, keepdims=True)
    scale = gm_ref[0:1] * lax.rsqrt(var + 1e-5)
    shift = bt_ref[0:1] - mean * scale
    o_ref[...] = jnp.maximum(g * scale.reshape(1, 1, DD)
                             + shift.reshape(1, 1, DD), 0.0)


_bnrelu_call = pl.pallas_call(
    _bnrelu_body,
    out_shape=jax.ShapeDtypeStruct((TT, N, DD), jnp.float32),
)


RB = 512
NRB = N // RB
_SCALE = 1.0 / float(DD) ** 0.5
_DEF = lax.Precision.DEFAULT


def _attn_body(h_ref, wq_ref, wk_ref, wv_ref, wagg_ref, al_ref, emb_ref,
               o_ref, xq, xk, xv, acc):
    hh = pl.program_id(1)
    x = h_ref[0]
    xq[...] = jnp.dot(x, wq_ref[...], preferred_element_type=jnp.float32,
                      precision=_HI) * _SCALE
    xk[...] = jnp.dot(x, wk_ref[...], preferred_element_type=jnp.float32,
                      precision=_HI)
    xv[...] = jnp.dot(x, wv_ref[...], preferred_element_type=jnp.float32,
                      precision=_HI)

    @pl.when(hh == 0)
    def _():
        acc[...] = jnp.zeros_like(acc)

    tri = jnp.where(
        lax.broadcasted_iota(jnp.int32, (RB, RB), 0)
        >= lax.broadcasted_iota(jnp.int32, (RB, RB), 1),
        0.0, -jnp.inf)

    def rb_body(i, _):
        qb = xq[pl.ds(i * RB, RB), :]

        def cb(j, st):
            m, l, o = st
            kb = xk[pl.ds(j * RB, RB), :]
            s = lax.dot_general(qb, kb, (((1,), (1,)), ((), ())),
                                preferred_element_type=jnp.float32,
                                precision=_DEF)
            m2 = jnp.maximum(m, jnp.max(s, axis=1, keepdims=True))
            corr = jnp.exp(m - m2)
            p = jnp.exp(s - m2)
            vb = xv[pl.ds(j * RB, RB), :]
            l2 = l * corr + jnp.sum(p, axis=1, keepdims=True)
            o2 = o * corr + jnp.dot(p, vb, preferred_element_type=jnp.float32,
                                    precision=_DEF)
            return (m2, l2, o2)

        init = (jnp.full((RB, 1), -jnp.inf, jnp.float32),
                jnp.zeros((RB, 1), jnp.float32),
                jnp.zeros((RB, DD), jnp.float32))
        m, l, o = lax.fori_loop(0, i, cb, init)
        kb = xk[pl.ds(i * RB, RB), :]
        s = lax.dot_general(qb, kb, (((1,), (1,)), ((), ())),
                            preferred_element_type=jnp.float32,
                            precision=_DEF) + tri
        m2 = jnp.maximum(m, jnp.max(s, axis=1, keepdims=True))
        corr = jnp.exp(m - m2)
        p = jnp.exp(s - m2)
        vb = xv[pl.ds(i * RB, RB), :]
        l = l * corr + jnp.sum(p, axis=1, keepdims=True)
        o = o * corr + jnp.dot(p, vb, preferred_element_type=jnp.float32,
                               precision=_DEF)
        acc[pl.ds(i * RB, RB), :] += o / l
        return 0

    lax.fori_loop(0, NRB, rb_body, 0)

    @pl.when(hh == NH - 1)
    def _():
        mh = acc[...] * (1.0 / NH)
        agg = jnp.dot(mh, wagg_ref[...], preferred_element_type=jnp.float32,
                      precision=_HI)
        a = al_ref[...]
        o_ref[0] = (1.0 - a) * emb_ref[...] + a * agg


_attn_call = pl.pallas_call(
    _attn_body,
    grid=(TT, NH),
    in_specs=[
        pl.BlockSpec((1, N, DD), lambda t, h: (t, 0, 0)),
        pl.BlockSpec((DD, DD), lambda t, h: (0, h)),
        pl.BlockSpec((DD, DD), lambda t, h: (0, h)),
        pl.BlockSpec((DD, DD), lambda t, h: (0, h)),
        pl.BlockSpec((DD, DD), lambda t, h: (0, 0)),
        pl.BlockSpec((N, DD), lambda t, h: (0, 0)),
        pl.BlockSpec((N, DD), lambda t, h: (0, 0)),
    ],
    out_specs=pl.BlockSpec((1, N, DD), lambda t, h: (t, 0, 0)),
    out_shape=jax.ShapeDtypeStruct((TT, N, DD), jnp.float32),
    scratch_shapes=[
        pltpu.VMEM((N, DD), jnp.float32),
        pltpu.VMEM((N, DD), jnp.float32),
        pltpu.VMEM((N, DD), jnp.float32),
        pltpu.VMEM((N, DD), jnp.float32),
    ],
)


def kernel(X, edge_index, edge_weight, emb_table, gcn1_W, gcn1_b, bn1_gamma,
           bn1_beta, gcn2_W, gcn2_b, bn2_gamma, bn2_beta, Wq, Wk, Wv, Wagg,
           alpha):
    A = _adj(edge_index.astype(jnp.int32), edge_weight).reshape(N, N)

    def row8(v):
        return jnp.broadcast_to(v.reshape(1, DD), (8, DD))

    dinv = _dinv_call(A)
    hs1 = _hs_call(X, gcn1_W, dinv)
    g1 = _gcnmm_call(A, hs1, hs1, dinv, row8(gcn1_b))
    hs2 = _bnhs_call(g1, row8(bn1_gamma), row8(bn1_beta), gcn2_W, dinv)
    g2 = _gcnmm_call(A, hs2, hs2, dinv, row8(gcn2_b))
    h2 = _bnrelu_call(g2, row8(bn2_gamma), row8(bn2_beta))

    alpha2 = jnp.broadcast_to(alpha, (N, DD))
    out = _attn_call(h2, Wq, Wk, Wv, Wagg, alpha2, emb_table)
    return out


# speedup vs baseline: 127.0037x; 2.0269x over previous
import functools

import jax
import jax.numpy as jnp
from jax import lax
from jax.experimental import pallas as pl
from jax.experimental.pallas import tpu as pltpu
from jax.experimental.pallas import tpu_sc as plsc

TT = 4
N = 2048
DD = 128
NH = 4
EE = 131072


QROWS = 512
QWORDS = QROWS * N
EPT = EE // 16
ZWORDS = 16384
TILE_QW = QWORDS // 16


def _make_adj():
    mesh = plsc.VectorSubcoreMesh(core_axis_name="c", subcore_axis_name="s")

    @functools.partial(
        pl.kernel,
        mesh=mesh,
        out_type=jax.ShapeDtypeStruct((N, N), jnp.float32),
        scratch_types=[
            pltpu.VMEM((EPT,), jnp.int32),
            pltpu.VMEM((EPT,), jnp.int32),
            pltpu.VMEM((EPT,), jnp.float32),
            pltpu.VMEM((EPT,), jnp.int32),
            pltpu.VMEM((EPT,), jnp.float32),
            pltpu.VMEM((ZWORDS,), jnp.float32),
            pltpu.VMEM_SHARED((QWORDS,), jnp.float32),
            pltpu.SemaphoreType.DMA,
        ],
    )
    def adj(edge_hbm, ew_hbm, out_hbm, r_v, c_v, ew_v, idx_v, val_v, zero_v,
            qmem, sem):
        cid = lax.axis_index("c")
        sid = lax.axis_index("s")
        ebase = sid * EPT
        pltpu.sync_copy(edge_hbm.at[0, pl.ds(ebase, EPT)], r_v)
        pltpu.sync_copy(edge_hbm.at[1, pl.ds(ebase, EPT)], c_v)
        pltpu.sync_copy(ew_hbm.at[pl.ds(ebase, EPT)], ew_v)

        def zbody(i, _):
            zero_v[pl.ds(i * 16, 16)] = jnp.zeros((16,), jnp.float32)
            return 0
        lax.fori_loop(0, ZWORDS // 16, zbody, 0)

        for q in range(2):
            q0 = (cid * 2 + q) * QROWS
            for z in range(TILE_QW // ZWORDS):
                pltpu.sync_copy(
                    zero_v, qmem.at[pl.ds(sid * TILE_QW + z * ZWORDS, ZWORDS)])
            plsc.subcore_barrier()

            def ebody(j, _):
                for kk in range(8):
                    sl = pl.ds(j * 128 + kk * 16, 16)
                    flat = c_v[sl] * N + r_v[sl]
                    rel = c_v[sl] - q0
                    ok = (rel >= 0) & (rel < QROWS)
                    idx_v[sl] = jnp.where(ok, rel * N + r_v[sl],
                                          flat & (QWORDS - 1))
                    val_v[sl] = jnp.where(ok, ew_v[sl], 0.0)
                return 0
            lax.fori_loop(0, EPT // 128, ebody, 0)
            pltpu.sync_copy(val_v, qmem.at[idx_v], add=True)
            plsc.subcore_barrier()
            cps = [
                pltpu.async_copy(
                    qmem.at[pl.ds((sid * 32 + k) * N, N)],
                    out_hbm.at[q0 + sid * 32 + k], sem)
                for k in range(32)
            ]
            for cp in cps:
                cp.wait()
            plsc.subcore_barrier()

    return adj


_adj = _make_adj()


_HI = lax.Precision.HIGHEST
_DEF = lax.Precision.DEFAULT


def _dinv_body(a_ref, x_ref, w_ref, o_ref, ab_ref, hs_ref):
    a = a_ref[...]
    s = jnp.sum(a, axis=1, keepdims=True) + 1.0
    dv = lax.rsqrt(s)
    o_ref[...] = jnp.broadcast_to(dv, (N // 8, DD))
    ab_ref[...] = a.astype(jnp.bfloat16)
    for t in range(TT):
        hs_ref[t] = (jnp.dot(x_ref[t], w_ref[...],
                             preferred_element_type=jnp.float32,
                             precision=_DEF) * dv).astype(jnp.bfloat16)


_dinv_call = pl.pallas_call(
    _dinv_body,
    grid=(8,),
    in_specs=[pl.BlockSpec((N // 8, N), lambda i: (i, 0)),
              pl.BlockSpec((TT, N // 8, DD), lambda i: (0, i, 0)),
              pl.BlockSpec((DD, DD), lambda i: (0, 0))],
    out_specs=[pl.BlockSpec((N // 8, DD), lambda i: (i, 0)),
               pl.BlockSpec((N // 8, N), lambda i: (i, 0)),
               pl.BlockSpec((TT, N // 8, DD), lambda i: (0, i, 0))],
    out_shape=[jax.ShapeDtypeStruct((N, DD), jnp.float32),
               jax.ShapeDtypeStruct((N, N), jnp.bfloat16),
               jax.ShapeDtypeStruct((TT, N, DD), jnp.bfloat16)],
)


def _bnhs_body(g_ref, gm_ref, bt_ref, w_ref, dv_ref, o_ref):
    g2 = g_ref[...].reshape(TT * N, DD)
    mean = jnp.mean(g2, axis=0, keepdims=True)
    var = jnp.mean((g2 - mean) ** 2, axis=0, keepdims=True)
    scale = gm_ref[0:1] * lax.rsqrt(var + 1e-5)
    shift = bt_ref[0:1] - mean * scale
    dv = dv_ref[...]
    for t in range(TT):
        h = jnp.maximum(g_ref[t] * scale + shift, 0.0)
        o_ref[t] = (jnp.dot(h, w_ref[...], preferred_element_type=jnp.float32,
                            precision=_DEF) * dv).astype(jnp.bfloat16)


_bnhs_call = pl.pallas_call(
    _bnhs_body,
    out_shape=jax.ShapeDtypeStruct((TT, N, DD), jnp.bfloat16),
)


_RBG = 256


def _gcnmm_body(a_ref, hs_ref, hsb_ref, dvb_ref, b_ref, o_ref):
    dvb = dvb_ref[...]
    for t in range(TT):
        m = (jnp.dot(a_ref[...], hs_ref[t], preferred_element_type=jnp.float32,
                     precision=_DEF)
             + hsb_ref[t].astype(jnp.float32))
        o_ref[t] = m * dvb + b_ref[0:1]


_gcnmm_call = pl.pallas_call(
    _gcnmm_body,
    grid=(N // _RBG,),
    in_specs=[
        pl.BlockSpec((_RBG, N), lambda i: (i, 0)),
        pl.BlockSpec((TT, N, DD), lambda i: (0, 0, 0)),
        pl.BlockSpec((TT, _RBG, DD), lambda i: (0, i, 0)),
        pl.BlockSpec((_RBG, DD), lambda i: (i, 0)),
        pl.BlockSpec((8, DD), lambda i: (0, 0)),
    ],
    out_specs=pl.BlockSpec((TT, _RBG, DD), lambda i: (0, i, 0)),
    out_shape=jax.ShapeDtypeStruct((TT, N, DD), jnp.float32),
)


def _bnrelu_body(g_ref, gm_ref, bt_ref, o_ref):
    g = g_ref[...]
    g2 = g.reshape(TT * N, DD)
    mean = jnp.mean(g2, axis=0, keepdims=True)
    var = jnp.mean((g2 - mean) ** 2, axis=0, keepdims=True)
    scale = gm_ref[0:1] * lax.rsqrt(var + 1e-5)
    shift = bt_ref[0:1] - mean * scale
    o_ref[...] = jnp.maximum(g * scale.reshape(1, 1, DD)
                             + shift.reshape(1, 1, DD), 0.0)


_bnrelu_call = pl.pallas_call(
    _bnrelu_body,
    out_shape=jax.ShapeDtypeStruct((TT, N, DD), jnp.float32),
)


RB = 512
NRB = N // RB
_SCALE = 1.4426950408889634 / float(DD) ** 0.5


def _attn_body(h_ref, wq_ref, wk_ref, wv_ref, wagg_ref, al_ref, emb_ref,
               o_ref, xq, xk, xv, acc):
    hh = pl.program_id(1)
    x = h_ref[0]
    xq[...] = (jnp.dot(x, wq_ref[...], preferred_element_type=jnp.float32,
                       precision=_DEF) * _SCALE).astype(jnp.bfloat16)
    xk[...] = jnp.dot(x, wk_ref[...], preferred_element_type=jnp.float32,
                      precision=_DEF).astype(jnp.bfloat16)
    xv[...] = jnp.dot(x, wv_ref[...], preferred_element_type=jnp.float32,
                      precision=_DEF).astype(jnp.bfloat16)

    @pl.when(hh == 0)
    def _():
        acc[...] = jnp.zeros_like(acc)

    tri = jnp.where(
        lax.broadcasted_iota(jnp.int32, (RB, RB), 0)
        >= lax.broadcasted_iota(jnp.int32, (RB, RB), 1),
        0.0, -jnp.inf)

    for i in range(NRB):
        qb = xq[i * RB:(i + 1) * RB, :]
        m = l = o = None
        for j in range(i + 1):
            kb = xk[j * RB:(j + 1) * RB, :]
            s = lax.dot_general(qb, kb, (((1,), (1,)), ((), ())),
                                preferred_element_type=jnp.float32,
                                precision=_DEF)
            if j == i:
                s = s + tri
            vb = xv[j * RB:(j + 1) * RB, :]
            if j == 0:
                m = jnp.max(s, axis=1, keepdims=True)
                p = jnp.exp2(s - m)
                l = jnp.sum(p, axis=1, keepdims=True)
                o = jnp.dot(p.astype(jnp.bfloat16), vb,
                            preferred_element_type=jnp.float32,
                            precision=_DEF)
            else:
                m2 = jnp.maximum(m, jnp.max(s, axis=1, keepdims=True))
                corr = jnp.exp2(m - m2)
                p = jnp.exp2(s - m2)
                l = l * corr + jnp.sum(p, axis=1, keepdims=True)
                o = o * corr + jnp.dot(p.astype(jnp.bfloat16), vb,
                                       preferred_element_type=jnp.float32,
                                       precision=_DEF)
                m = m2
        acc[i * RB:(i + 1) * RB, :] += o * (1.0 / l)

    @pl.when(hh == NH - 1)
    def _():
        mh = acc[...] * (1.0 / NH)
        agg = jnp.dot(mh, wagg_ref[...], preferred_element_type=jnp.float32,
                      precision=_DEF)
        a = al_ref[...]
        o_ref[0] = (1.0 - a) * emb_ref[...] + a * agg


_attn_call = pl.pallas_call(
    _attn_body,
    grid=(TT, NH),
    in_specs=[
        pl.BlockSpec((1, N, DD), lambda t, h: (t, 0, 0)),
        pl.BlockSpec((DD, DD), lambda t, h: (0, h)),
        pl.BlockSpec((DD, DD), lambda t, h: (0, h)),
        pl.BlockSpec((DD, DD), lambda t, h: (0, h)),
        pl.BlockSpec((DD, DD), lambda t, h: (0, 0)),
        pl.BlockSpec((N, DD), lambda t, h: (0, 0)),
        pl.BlockSpec((N, DD), lambda t, h: (0, 0)),
    ],
    out_specs=pl.BlockSpec((1, N, DD), lambda t, h: (t, 0, 0)),
    out_shape=jax.ShapeDtypeStruct((TT, N, DD), jnp.float32),
    scratch_shapes=[
        pltpu.VMEM((N, DD), jnp.bfloat16),
        pltpu.VMEM((N, DD), jnp.bfloat16),
        pltpu.VMEM((N, DD), jnp.bfloat16),
        pltpu.VMEM((N, DD), jnp.float32),
    ],
)


def kernel(X, edge_index, edge_weight, emb_table, gcn1_W, gcn1_b, bn1_gamma,
           bn1_beta, gcn2_W, gcn2_b, bn2_gamma, bn2_beta, Wq, Wk, Wv, Wagg,
           alpha):
    A = _adj(edge_index.astype(jnp.int32), edge_weight)

    def row8(v):
        return jnp.broadcast_to(v.reshape(1, DD), (8, DD))

    dinv, A_bf, hs1 = _dinv_call(A, X, gcn1_W)
    g1 = _gcnmm_call(A_bf, hs1, hs1, dinv, row8(gcn1_b))
    hs2 = _bnhs_call(g1, row8(bn1_gamma), row8(bn1_beta), gcn2_W, dinv)
    g2 = _gcnmm_call(A_bf, hs2, hs2, dinv, row8(gcn2_b))
    h2 = _bnrelu_call(g2, row8(bn2_gamma), row8(bn2_beta))

    alpha2 = jnp.broadcast_to(alpha, (N, DD))
    out = _attn_call(h2, Wq, Wk, Wv, Wagg, alpha2, emb_table)
    return out
